# baseline (speedup 1.0000x reference)
import functools

import jax
import jax.numpy as jnp
from jax.experimental import pallas as pl

K = 3
ROWS = 3136


def _combine_body(agg_ref, inv_ref, h_ref, r_ref, b_ref, o_ref):
    x = agg_ref[...] * inv_ref[...] + h_ref[...] @ r_ref[...] + b_ref[...]
    o_ref[...] = jnp.where(x > 0, x, jnp.exp(jnp.minimum(x, 0.0)) - 1.0)


def _combine(agg, invcnt, h, r, b):
    n, cout = agg.shape
    cin = h.shape[1]
    grid = (n // ROWS,)
    return pl.pallas_call(
        _combine_body,
        grid=grid,
        in_specs=[
            pl.BlockSpec((ROWS, cout), lambda i: (i, 0)),
            pl.BlockSpec((ROWS, 1), lambda i: (i, 0)),
            pl.BlockSpec((ROWS, cin), lambda i: (i, 0)),
            pl.BlockSpec((cin, cout), lambda i: (0, 0)),
            pl.BlockSpec((1, cout), lambda i: (0, 0)),
        ],
        out_specs=pl.BlockSpec((ROWS, cout), lambda i: (i, 0)),
        out_shape=jax.ShapeDtypeStruct((n, cout), jnp.float32),
    )(agg, invcnt, h, r, b)


def kernel(x, edge_index, edge_attr, y, params):
    src = edge_index[0]
    dst = edge_index[1]
    n = x.shape[0]

    v = edge_attr * (K - 1)
    lo = jnp.clip(jnp.floor(v), 0.0, K - 2)
    frac = v - lo
    loi = lo.astype(jnp.int32)
    b0 = jnp.stack([1.0 - frac[:, 0], frac[:, 0]], 1)
    b1 = jnp.stack([1.0 - frac[:, 1], frac[:, 1]], 1)
    k0 = jnp.stack([loi[:, 0], loi[:, 0] + 1], 1)
    k1 = jnp.stack([loi[:, 1], loi[:, 1] + 1], 1)
    basis = (b0[:, :, None] * b1[:, None, :]).reshape(-1, 4)
    widx = (k0[:, :, None] + K * k1[:, None, :]).reshape(-1, 4)

    cnt = jax.ops.segment_sum(jnp.ones((dst.shape[0],), x.dtype), dst,
                              num_segments=n)
    invcnt = (1.0 / jnp.maximum(cnt, 1.0))[:, None]

    h = x
    for (w, r, b) in params:
        xk = jnp.einsum('ni,kio->nko', h, w)
        msg = jnp.zeros((src.shape[0], w.shape[2]), h.dtype)
        for s in range(4):
            msg = msg + basis[:, s:s + 1] * xk[src, widx[:, s]]
        agg = jax.ops.segment_sum(msg, dst, num_segments=n)
        h = _combine(agg, invcnt, h, r, b[None, :])
    return h.reshape(1, y.shape[-1], y.shape[-1])
